# Initial kernel scaffold; baseline (speedup 1.0000x reference)
import jax
import jax.numpy as jnp
from jax.experimental import pallas as pl


def kernel(scores):
    raise NotImplementedError("write your pallas kernel here")


# speedup vs baseline: 1.8184x; 1.8184x over previous
import jax
import jax.numpy as jnp
import numpy as np
from jax.experimental import pallas as pl

_EPSILON = float(np.finfo(np.float32).tiny)
_K = 32
_TAU = 0.1

_ROW_BLOCK = 16


def _gumbel_topk_kernel(x_ref, g_ref, out_ref):
    s = x_ref[...] + g_ref[...]
    khot = jnp.zeros_like(s)
    onehot = jnp.zeros_like(s)

    def soft_body(_, carry):
        s, khot, onehot = carry
        khot_mask = jnp.maximum(1.0 - onehot, _EPSILON)
        s = s + jnp.log(khot_mask)
        t = s / _TAU
        m = jnp.max(t, axis=1, keepdims=True)
        e = jnp.exp(t - m)
        onehot = e / jnp.sum(e, axis=1, keepdims=True)
        khot = khot + onehot
        return (s, khot, onehot)

    s, khot, onehot = jax.lax.fori_loop(
        0, _K, soft_body, (s, khot, onehot), unroll=False
    )

    iota = jax.lax.broadcasted_iota(jnp.int32, khot.shape, 1)
    big = jnp.int32(np.iinfo(np.int32).max)

    def topk_body(_, carry):
        w, hard = carry
        m = jnp.max(w, axis=1, keepdims=True)
        idx = jnp.min(jnp.where(w == m, iota, big), axis=1, keepdims=True)
        sel = iota == idx
        hard = jnp.where(sel, 1.0, hard)
        w = jnp.where(sel, -jnp.inf, w)
        return (w, hard)

    _, hard = jax.lax.fori_loop(
        0, _K, topk_body, (khot, jnp.zeros_like(khot)), unroll=False
    )

    out_ref[...] = (hard - khot) + khot


def kernel(scores):
    bsz, nmax, _, ensemble = scores.shape
    rows = bsz * ensemble
    cols = nmax * nmax
    x = jnp.transpose(scores, (0, 3, 1, 2)).reshape(rows, cols)
    g = jax.random.gumbel(jax.random.key(42), x.shape, dtype=x.dtype)

    res = pl.pallas_call(
        _gumbel_topk_kernel,
        grid=(rows // _ROW_BLOCK,),
        in_specs=[
            pl.BlockSpec((_ROW_BLOCK, cols), lambda i: (i, 0)),
            pl.BlockSpec((_ROW_BLOCK, cols), lambda i: (i, 0)),
        ],
        out_specs=pl.BlockSpec((_ROW_BLOCK, cols), lambda i: (i, 0)),
        out_shape=jax.ShapeDtypeStruct((rows, cols), x.dtype),
    )(x, g)

    res = res.reshape(bsz, ensemble, nmax, nmax)
    return jnp.transpose(res, (0, 2, 3, 1))
